# baseline (speedup 1.0000x reference)
import functools

import jax
import jax.numpy as jnp
import numpy as np
from jax import lax
from jax.experimental import pallas as pl
from jax.experimental.pallas import tpu as pltpu
from jax.experimental.pallas import tpu_sc as plsc

_B = 16384
_E = 32
_NNEG = 15
_REPS = _NNEG + 1
_OUT = _B * _REPS
_NW = 32
_HALF = _B // 2
_CHUNK = 1024


def _shift_table():
    rng = np.random.default_rng(0)
    picks = rng.choice(np.arange(1, _B), size=_NNEG, replace=False)
    return [0] + [int(a) for a in picks]


_SHIFTS = _shift_table()


def _worker_chunks(w):
    k, half = divmod(w, 2)
    i0 = half * _HALF
    dst = k * _B + i0
    src = (_SHIFTS[k] + i0) % _B
    n1 = min(_HALF, _B - src)
    segs = [(True, i0, dst, _HALF), (False, src, dst, n1)]
    if n1 < _HALF:
        segs.append((False, 0, dst + n1, _HALF - n1))
    chunks = []
    for isq, s, d, n in segs:
        off = 0
        while off < n:
            c = min(_CHUNK, n - off)
            chunks.append((isq, s + off, d + off, c))
            off += c
    return chunks


_mesh = plsc.VectorSubcoreMesh(core_axis_name="c", subcore_axis_name="s")


@functools.partial(
    pl.kernel,
    out_type=(
        jax.ShapeDtypeStruct((_OUT * _E,), jnp.float32),
        jax.ShapeDtypeStruct((_OUT * _E,), jnp.float32),
    ),
    mesh=_mesh,
    scratch_types=[
        pltpu.VMEM((_CHUNK * _E,), jnp.float32),
        pltpu.VMEM((_CHUNK * _E,), jnp.float32),
        pltpu.SemaphoreType.DMA,
        pltpu.SemaphoreType.DMA,
        pltpu.SemaphoreType.DMA,
        pltpu.SemaphoreType.DMA,
    ],
)
def _sc_sample(q_hbm, it_hbm, qout_hbm, iout_hbm, b0, b1, si0, si1, so0, so1):
    wid = lax.axis_index("s") * 2 + lax.axis_index("c")
    bufs = (b0, b1)
    sin = (si0, si1)
    sout = (so0, so1)
    for w in range(_NW):
        chunks = _worker_chunks(w)

        @pl.when(wid == w)
        def _(chunks=chunks):
            in_cp = [None, None]
            out_cp = [None, None]

            def issue_in(idx):
                isq, s, _d, n = chunks[idx]
                src_ref = q_hbm if isq else it_hbm
                b = idx % 2
                in_cp[b] = pltpu.async_copy(
                    src_ref.at[pl.ds(s * _E, n * _E)],
                    bufs[b].at[pl.ds(0, n * _E)],
                    sin[b],
                )

            issue_in(0)
            for idx, (isq, _s, d, n) in enumerate(chunks):
                b = idx % 2
                if idx + 1 < len(chunks):
                    b2 = (idx + 1) % 2
                    if out_cp[b2] is not None:
                        out_cp[b2].wait()
                    issue_in(idx + 1)
                in_cp[b].wait()
                dst_ref = qout_hbm if isq else iout_hbm
                out_cp[b] = pltpu.async_copy(
                    bufs[b].at[pl.ds(0, n * _E)],
                    dst_ref.at[pl.ds(d * _E, n * _E)],
                    sout[b],
                )
            for b in range(2):
                if out_cp[b] is not None:
                    out_cp[b].wait()


def kernel(query_embeddings, item_embeddings):
    q_out, it_out = _sc_sample(
        query_embeddings.reshape(-1), item_embeddings.reshape(-1)
    )
    return q_out.reshape(_OUT, _E), it_out.reshape(_OUT, _E)


# speedup vs baseline: 1.0098x; 1.0098x over previous
import functools

import jax
import jax.numpy as jnp
import numpy as np
from jax import lax
from jax.experimental import pallas as pl
from jax.experimental.pallas import tpu as pltpu
from jax.experimental.pallas import tpu_sc as plsc

_B = 16384
_E = 32
_NNEG = 15
_REPS = _NNEG + 1
_OUT = _B * _REPS
_NW = 32
_HALF = _B // 2
_CHUNK = 2032


def _shift_table():
    rng = np.random.default_rng(0)
    picks = rng.choice(np.arange(1, _B), size=_NNEG, replace=False)
    return [0] + [int(a) for a in picks]


_SHIFTS = _shift_table()


def _worker_chunks(w):
    k, half = divmod(w, 2)
    i0 = half * _HALF
    dst = k * _B + i0
    src = (_SHIFTS[k] + i0) % _B
    n1 = min(_HALF, _B - src)
    segs = [(True, i0, dst, _HALF), (False, src, dst, n1)]
    if n1 < _HALF:
        segs.append((False, 0, dst + n1, _HALF - n1))
    chunks = []
    for isq, s, d, n in segs:
        off = 0
        while off < n:
            c = min(_CHUNK, n - off)
            chunks.append((isq, s + off, d + off, c))
            off += c
    return chunks


_mesh = plsc.VectorSubcoreMesh(core_axis_name="c", subcore_axis_name="s")


@functools.partial(
    pl.kernel,
    out_type=(
        jax.ShapeDtypeStruct((_OUT * _E,), jnp.float32),
        jax.ShapeDtypeStruct((_OUT * _E,), jnp.float32),
    ),
    mesh=_mesh,
    scratch_types=[
        pltpu.VMEM((_CHUNK * _E,), jnp.float32),
        pltpu.VMEM((_CHUNK * _E,), jnp.float32),
        pltpu.SemaphoreType.DMA,
        pltpu.SemaphoreType.DMA,
        pltpu.SemaphoreType.DMA,
        pltpu.SemaphoreType.DMA,
    ],
)
def _sc_sample(q_hbm, it_hbm, qout_hbm, iout_hbm, b0, b1, si0, si1, so0, so1):
    wid = lax.axis_index("s") * 2 + lax.axis_index("c")
    bufs = (b0, b1)
    sin = (si0, si1)
    sout = (so0, so1)
    for w in range(_NW):
        chunks = _worker_chunks(w)

        @pl.when(wid == w)
        def _(chunks=chunks):
            in_cp = [None, None]
            out_cp = [None, None]

            def issue_in(idx):
                isq, s, _d, n = chunks[idx]
                src_ref = q_hbm if isq else it_hbm
                b = idx % 2
                in_cp[b] = pltpu.async_copy(
                    src_ref.at[pl.ds(s * _E, n * _E)],
                    bufs[b].at[pl.ds(0, n * _E)],
                    sin[b],
                )

            issue_in(0)
            for idx, (isq, _s, d, n) in enumerate(chunks):
                b = idx % 2
                if idx + 1 < len(chunks):
                    b2 = (idx + 1) % 2
                    if out_cp[b2] is not None:
                        out_cp[b2].wait()
                    issue_in(idx + 1)
                in_cp[b].wait()
                dst_ref = qout_hbm if isq else iout_hbm
                out_cp[b] = pltpu.async_copy(
                    bufs[b].at[pl.ds(0, n * _E)],
                    dst_ref.at[pl.ds(d * _E, n * _E)],
                    sout[b],
                )
            for b in range(2):
                if out_cp[b] is not None:
                    out_cp[b].wait()


def kernel(query_embeddings, item_embeddings):
    q_out, it_out = _sc_sample(
        query_embeddings.reshape(-1), item_embeddings.reshape(-1)
    )
    return q_out.reshape(_OUT, _E), it_out.reshape(_OUT, _E)


# speedup vs baseline: 1.1974x; 1.1858x over previous
import functools

import jax
import jax.numpy as jnp
import numpy as np
from jax import lax
from jax.experimental import pallas as pl
from jax.experimental.pallas import tpu as pltpu
from jax.experimental.pallas import tpu_sc as plsc

_B = 16384
_E = 32
_NNEG = 15
_REPS = _NNEG + 1
_OUT = _B * _REPS
_NW = 32
_EI = _B // 8
_EIW = _EI * _E


def _shift_table():
    rng = np.random.default_rng(0)
    picks = rng.choice(np.arange(1, _B), size=_NNEG, replace=False)
    return [0] + [int(a) for a in picks]


_SHIFTS = _shift_table()

_mesh = plsc.VectorSubcoreMesh(core_axis_name="c", subcore_axis_name="s")


@functools.partial(
    pl.kernel,
    out_type=jax.ShapeDtypeStruct((_OUT * _E,), jnp.float32),
    mesh=_mesh,
    scratch_types=[pltpu.VMEM((_EIW,), jnp.float32)]
    + [pltpu.SemaphoreType.DMA for _ in range(4)],
)
def _sc_tile_query(q_hbm, qout_hbm, buf, *sems):
    wid = lax.axis_index("s") * 2 + lax.axis_index("c")
    for w in range(_NW):
        e, g = w % 8, w // 8

        @pl.when(wid == w)
        def _(e=e, g=g):
            pltpu.sync_copy(q_hbm.at[pl.ds(e * _EIW, _EIW)], buf)
            cps = []
            for j in range(4):
                k = g * 4 + j
                cps.append(
                    pltpu.async_copy(
                        buf,
                        qout_hbm.at[pl.ds((k * _B + e * _EI) * _E, _EIW)],
                        sems[j],
                    )
                )
            for c in cps:
                c.wait()


def _tc_body(item_ref, shifts_ref, out_ref, scratch_ref):
    k = pl.program_id(0)

    @pl.when(k == 0)
    def _():
        scratch_ref[pl.ds(0, _B), :] = item_ref[...]
        scratch_ref[pl.ds(_B, _B), :] = item_ref[...]

    a = shifts_ref[k]
    out_ref[...] = scratch_ref[pl.ds(a, _B), :]


def _tc_roll_items(item):
    shifts = jnp.asarray(_SHIFTS, jnp.int32)
    return pl.pallas_call(
        _tc_body,
        grid=(_REPS,),
        in_specs=[
            pl.BlockSpec((_B, _E), lambda k: (0, 0)),
            pl.BlockSpec(memory_space=pltpu.SMEM),
        ],
        out_specs=pl.BlockSpec((_B, _E), lambda k: (k, 0)),
        out_shape=jax.ShapeDtypeStruct((_OUT, _E), jnp.float32),
        scratch_shapes=[pltpu.VMEM((2 * _B, _E), jnp.float32)],
    )(item, shifts)


def kernel(query_embeddings, item_embeddings):
    q_out = _sc_tile_query(query_embeddings.reshape(-1))
    it_out = _tc_roll_items(item_embeddings)
    return q_out.reshape(_OUT, _E), it_out


# speedup vs baseline: 7.4328x; 6.2072x over previous
import functools

import jax
import jax.numpy as jnp
import numpy as np
from jax import lax
from jax.experimental import pallas as pl
from jax.experimental.pallas import tpu as pltpu
from jax.experimental.pallas import tpu_sc as plsc

_B = 16384
_E = 32
_NNEG = 15
_REPS = _NNEG + 1
_OUT = _B * _REPS
_NW = 32
_EI = _B // 8


def _shift_table():
    rng = np.random.default_rng(0)
    picks = rng.choice(np.arange(1, _B), size=_NNEG, replace=False)
    return [0] + [int(a) for a in picks]


_SHIFTS = _shift_table()

_mesh = plsc.VectorSubcoreMesh(core_axis_name="c", subcore_axis_name="s")


@functools.partial(
    pl.kernel,
    out_type=jax.ShapeDtypeStruct((_E, _OUT), jnp.float32),
    mesh=_mesh,
    scratch_types=[pltpu.VMEM((_E, _EI), jnp.float32)]
    + [pltpu.SemaphoreType.DMA for _ in range(4)],
)
def _sc_tile_query(qt_hbm, qout_hbm, buf, *sems):
    wid = lax.axis_index("s") * 2 + lax.axis_index("c")
    for w in range(_NW):
        e, g = w % 8, w // 8

        @pl.when(wid == w)
        def _(e=e, g=g):
            pltpu.sync_copy(qt_hbm.at[:, pl.ds(e * _EI, _EI)], buf)
            cps = []
            for j in range(4):
                k = g * 4 + j
                cps.append(
                    pltpu.async_copy(
                        buf,
                        qout_hbm.at[:, pl.ds(k * _B + e * _EI, _EI)],
                        sems[j],
                    )
                )
            for c in cps:
                c.wait()


def _tc_body(item_ref, shifts_ref, out_ref, scratch_ref):
    k = pl.program_id(0)

    @pl.when(k == 0)
    def _():
        scratch_ref[:, pl.ds(0, _B)] = item_ref[...]
        scratch_ref[:, pl.ds(_B, _B)] = item_ref[...]

    a = shifts_ref[k]
    a_hi = pl.multiple_of((a // 128) * 128, 128)
    r = a - a_hi
    coarse = scratch_ref[:, pl.ds(a_hi, _B)]
    out_ref[...] = pltpu.roll(coarse, (_B - r) % _B, 1)


def _tc_roll_items(item_t):
    shifts = jnp.asarray(_SHIFTS, jnp.int32)
    return pl.pallas_call(
        _tc_body,
        grid=(_REPS,),
        in_specs=[
            pl.BlockSpec((_E, _B), lambda k: (0, 0)),
            pl.BlockSpec(memory_space=pltpu.SMEM),
        ],
        out_specs=pl.BlockSpec((_E, _B), lambda k: (0, k)),
        out_shape=jax.ShapeDtypeStruct((_E, _OUT), jnp.float32),
        scratch_shapes=[pltpu.VMEM((_E, 2 * _B), jnp.float32)],
    )(item_t, shifts)


def kernel(query_embeddings, item_embeddings):
    q_out_t = _sc_tile_query(query_embeddings.T)
    it_out_t = _tc_roll_items(item_embeddings.T)
    return q_out_t.T, it_out_t.T
